# baseline (speedup 1.0000x reference)
import functools

import jax
import jax.numpy as jnp
from jax import lax
from jax.experimental import pallas as pl
from jax.experimental.pallas import tpu as pltpu
from jax.experimental.pallas import tpu_sc as plsc

_N = 64
_C = 100000
_K = 8
_INV_T = 0.5
_NB = 8
_BLK = 12800
_NEG = float("-inf")
_IMAX = 2**31 - 1
_W = 128
_NWIN = (_C + _W - 1) // _W


def _stage1(x_ref, outv_ref, outi_ref, sum_ref, runv_ref, runi_ref):
    b = pl.program_id(0)

    @pl.when(b == 0)
    def _init():
        sum_ref[...] = jnp.zeros_like(sum_ref)
        runv_ref[...] = jnp.full((_N, _K), _NEG, jnp.float32)
        runi_ref[...] = jnp.full((_N, _K), _IMAX, jnp.int32)

    x = x_ref[...]
    col = jax.lax.broadcasted_iota(jnp.int32, (_N, _BLK), 1) + b * _BLK
    x = jnp.where(col < _C, x, _NEG)
    e = jnp.exp(x * _INV_T)
    sum_ref[...] += jnp.sum(e, axis=1, keepdims=True)

    bvs, bis = [], []
    for i in range(_K):
        m = jnp.max(x, axis=1, keepdims=True)
        hit = x == m
        idx = jnp.min(jnp.where(hit, col, _IMAX), axis=1, keepdims=True)
        bvs.append(m)
        bis.append(idx)
        x = jnp.where(hit & (col == idx), _NEG, x)

    V = jnp.concatenate([runv_ref[...]] + bvs, axis=1)
    I = jnp.concatenate([runi_ref[...]] + bis, axis=1)
    nv, ni = [], []
    for i in range(_K):
        m = jnp.max(V, axis=1, keepdims=True)
        hit = V == m
        idx = jnp.min(jnp.where(hit, I, _IMAX), axis=1, keepdims=True)
        nv.append(m)
        ni.append(idx)
        V = jnp.where(hit & (I == idx), _NEG, V)
    runv_ref[...] = jnp.concatenate(nv, axis=1)
    runi_ref[...] = jnp.concatenate(ni, axis=1)

    @pl.when(b == _NB - 1)
    def _final():
        topv = runv_ref[...]
        denom = sum_ref[...]
        for i in range(_K):
            e = jnp.exp(topv[:, i:i + 1] * _INV_T)
            outv_ref[:, i:i + 1] = e / denom
            denom = denom - e
        outi_ref[...] = runi_ref[...]


_NW = 32
_RPW = _N // _NW
_CHUNK = 20000
_NCH = _C // _CHUNK
_VPC = _CHUNK // 16
_NSEM = 4


def _sc_zero_body(out_hbm, zbuf, *sems):
    wid = lax.axis_index("s") * 2 + lax.axis_index("c")
    r0 = wid * _RPW

    def _zero(i, carry):
        zbuf[pl.ds(i * 16, 16)] = jnp.zeros((16,), jnp.float32)
        return carry

    lax.fori_loop(0, _VPC, _zero, 0)

    pending = [None] * _NSEM
    for t, (r, c) in enumerate([(r, c) for r in range(_RPW)
                                for c in range(_NCH)]):
        b = t % _NSEM
        if pending[b] is not None:
            pending[b].wait()
        pending[b] = pltpu.async_copy(
            zbuf, out_hbm.at[r0 + r, pl.ds(c * _CHUNK, _CHUNK)], sems[b])
    for cp in pending:
        if cp is not None:
            cp.wait()


@functools.lru_cache(maxsize=1)
def _make_sc_zero():
    mesh = plsc.VectorSubcoreMesh(
        core_axis_name="c", subcore_axis_name="s", num_cores=2,
        num_subcores=16)
    return pl.kernel(
        _sc_zero_body,
        out_type=jax.ShapeDtypeStruct((_N, _C), jnp.float32),
        mesh=mesh,
        scratch_types=[pltpu.VMEM((_CHUNK,), jnp.float32)]
        + [pltpu.SemaphoreType.DMA] * _NSEM,
        compiler_params=pltpu.CompilerParams(
            use_tc_tiling_on_sc=False, needs_layout_passes=False),
    )


_RG = 8


def _scatter_body(idx_ref, v_ref, z_ref, o_ref):
    del z_ref
    s = pl.program_id(0)
    r = s // _K
    rg = (r // _RG) * _RG
    base = (idx_ref[r, s % _K] // _W) * _W
    lane = jax.lax.broadcasted_iota(jnp.int32, (_RG, _W), 1) + base
    sub = jax.lax.broadcasted_iota(jnp.int32, (_RG, _W), 0)
    vg = v_ref[pl.ds(rg, _RG), :]
    acc = jnp.zeros((_RG, _W), jnp.float32)
    for ss in range(_RG):
        for k in range(_K):
            hit = (sub == ss) & (lane == idx_ref[rg + ss, k])
            acc = jnp.where(hit, vg[ss:ss + 1, k:k + 1], acc)
    o_ref[...] = acc


def kernel(normu, cls):
    z = _make_sc_zero()()

    outv, outi = pl.pallas_call(
        _stage1,
        grid=(_NB,),
        in_specs=[pl.BlockSpec((_N, _BLK), lambda b: (0, b))],
        out_specs=[
            pl.BlockSpec((_N, _K), lambda b: (0, 0)),
            pl.BlockSpec((_N, _K), lambda b: (0, 0)),
        ],
        out_shape=[
            jax.ShapeDtypeStruct((_N, _K), jnp.float32),
            jax.ShapeDtypeStruct((_N, _K), jnp.int32),
        ],
        scratch_shapes=[
            pltpu.VMEM((_N, 1), jnp.float32),
            pltpu.VMEM((_N, _K), jnp.float32),
            pltpu.VMEM((_N, _K), jnp.int32),
        ],
    )(cls)

    classes = pl.pallas_call(
        _scatter_body,
        grid_spec=pltpu.PrefetchScalarGridSpec(
            num_scalar_prefetch=1,
            grid=(_N * _K,),
            in_specs=[
                pl.BlockSpec((_N, _K), lambda s, idx: (0, 0)),
                pl.BlockSpec(memory_space=pl.ANY),
            ],
            out_specs=pl.BlockSpec(
                (_RG, _W),
                lambda s, idx: (s // _K // _RG, idx[s // _K, s % _K] // _W)),
        ),
        out_shape=jax.ShapeDtypeStruct((_N, _C), jnp.float32),
        input_output_aliases={2: 0},
    )(outi, outv, z)

    return (normu, classes)


# speedup vs baseline: 2.0445x; 2.0445x over previous
import functools

import jax
import jax.numpy as jnp
from jax import lax
from jax.experimental import pallas as pl
from jax.experimental.pallas import tpu as pltpu
from jax.experimental.pallas import tpu_sc as plsc

_N = 64
_C = 100000
_K = 8
_INV_T = 0.5
_NB = 8
_BLK = 12800
_NEG = float("-inf")
_IMAX = 2**31 - 1
_WIN = 128
_G = _N // 8


_TAIL = _C % _WIN
_TSTART = _C - _TAIL


def _stage1(x_ref, outv_ref, outi_ref, tail_ref, sum_ref, runv_ref, runi_ref):
    b = pl.program_id(0)

    @pl.when(b == 0)
    def _init():
        sum_ref[...] = jnp.zeros_like(sum_ref)
        runv_ref[...] = jnp.full((_N, _K), _NEG, jnp.float32)
        runi_ref[...] = jnp.full((_N, _K), _IMAX, jnp.int32)

    x = x_ref[...]
    col = jax.lax.broadcasted_iota(jnp.int32, (_N, _BLK), 1) + b * _BLK
    x = jnp.where(col < _C, x, _NEG)
    e = jnp.exp(x * _INV_T)
    sum_ref[...] += jnp.sum(e, axis=1, keepdims=True)

    bvs, bis = [], []
    for i in range(_K):
        m = jnp.max(x, axis=1, keepdims=True)
        hit = x == m
        idx = jnp.min(jnp.where(hit, col, _IMAX), axis=1, keepdims=True)
        bvs.append(m)
        bis.append(idx)
        x = jnp.where(hit & (col == idx), _NEG, x)

    V = jnp.concatenate([runv_ref[...]] + bvs, axis=1)
    I = jnp.concatenate([runi_ref[...]] + bis, axis=1)
    nv, ni = [], []
    for i in range(_K):
        m = jnp.max(V, axis=1, keepdims=True)
        hit = V == m
        idx = jnp.min(jnp.where(hit, I, _IMAX), axis=1, keepdims=True)
        nv.append(m)
        ni.append(idx)
        V = jnp.where(hit & (I == idx), _NEG, V)
    runv_ref[...] = jnp.concatenate(nv, axis=1)
    runi_ref[...] = jnp.concatenate(ni, axis=1)

    @pl.when(b == _NB - 1)
    def _final():
        topv = runv_ref[...]
        topi = runi_ref[...]
        denom = sum_ref[...]
        vals = []
        for i in range(_K):
            e = jnp.exp(topv[:, i:i + 1] * _INV_T)
            vals.append(e / denom)
            outv_ref[:, i:i + 1] = vals[-1]
            denom = denom - e
        outi_ref[...] = topi
        jt = jax.lax.broadcasted_iota(jnp.int32, (_N, _TAIL), 1) + _TSTART
        tacc = jnp.zeros((_N, _TAIL), jnp.float32)
        for i in range(_K):
            tacc = jnp.where(topi[:, i:i + 1] == jt, vals[i], tacc)
        tail_ref[...] = tacc


_NW = 32
_RPW = _N // _NW
_CHUNK = 20000
_NCH = _C // _CHUNK
_VPC = _CHUNK // 16
_NSEM = 4


def _sc_zero_body(out_hbm, zbuf, *sems):
    wid = lax.axis_index("s") * 2 + lax.axis_index("c")
    r0 = wid * _RPW

    def _zero(i, carry):
        zbuf[pl.ds(i * 16, 16)] = jnp.zeros((16,), jnp.float32)
        return carry

    lax.fori_loop(0, _VPC, _zero, 0)

    pending = [None] * _NSEM
    for t, (r, c) in enumerate([(r, c) for r in range(_RPW)
                                for c in range(_NCH)]):
        b = t % _NSEM
        if pending[b] is not None:
            pending[b].wait()
        pending[b] = pltpu.async_copy(
            zbuf, out_hbm.at[r0 + r, pl.ds(c * _CHUNK, _CHUNK)], sems[b])
    for cp in pending:
        if cp is not None:
            cp.wait()


@functools.lru_cache(maxsize=1)
def _make_sc_zero():
    mesh = plsc.VectorSubcoreMesh(
        core_axis_name="c", subcore_axis_name="s", num_cores=2,
        num_subcores=16)
    return pl.kernel(
        _sc_zero_body,
        out_type=jax.ShapeDtypeStruct((_N, _C), jnp.float32),
        mesh=mesh,
        scratch_types=[pltpu.VMEM((_CHUNK,), jnp.float32)]
        + [pltpu.SemaphoreType.DMA] * _NSEM,
        compiler_params=pltpu.CompilerParams(
            use_tc_tiling_on_sc=False, needs_layout_passes=False),
    )


def _scatter_body(idx_ref, idxv_ref, v_ref, z_ref, o_ref,
                  bufa, bufb, sema, semb):
    del z_ref
    idxs = idxv_ref[...]
    vals = v_ref[...]
    jota = jax.lax.broadcasted_iota(jnp.int32, (8, _WIN), 1)

    def do_pair(i, buf, sem, wait_first):
        if wait_first:
            for g in range(_G):
                pltpu.make_async_copy(
                    buf.at[pl.ds(g * 8, 8), :],
                    o_ref.at[pl.ds(g * 8, 8), pl.ds(0, _WIN)],
                    sem).wait()
        copies = []
        for g in range(_G):
            base = jnp.minimum(idx_ref[g * 8 + i // _K, i % _K] // _WIN,
                               _C // _WIN - 1) * _WIN
            win = base + jota
            acc = jnp.zeros((8, _WIN), jnp.float32)
            for k2 in range(_K):
                hit = idxs[g * 8:(g + 1) * 8, k2:k2 + 1] == win
                acc = jnp.where(hit, vals[g * 8:(g + 1) * 8, k2:k2 + 1], acc)
            buf[pl.ds(g * 8, 8), :] = acc
            copies.append((g, base))
        for g, base in copies:
            pltpu.make_async_copy(
                buf.at[pl.ds(g * 8, 8), :],
                o_ref.at[pl.ds(g * 8, 8), pl.ds(base, _WIN)],
                sem).start()

    do_pair(0, bufa, sema, False)
    do_pair(1, bufb, semb, False)

    def step(j, carry):
        do_pair(2 * j, bufa, sema, True)
        do_pair(2 * j + 1, bufb, semb, True)
        return carry

    lax.fori_loop(1, (_K * 8) // 2, step, 0)

    for buf, sem in ((bufa, sema), (bufb, semb)):
        for g in range(_G):
            pltpu.make_async_copy(
                buf.at[pl.ds(g * 8, 8), :],
                o_ref.at[pl.ds(g * 8, 8), pl.ds(0, _WIN)],
                sem).wait()


def kernel(normu, cls):
    z = _make_sc_zero()()

    outv, outi, tail = pl.pallas_call(
        _stage1,
        grid=(_NB,),
        in_specs=[pl.BlockSpec((_N, _BLK), lambda b: (0, b))],
        out_specs=[
            pl.BlockSpec((_N, _K), lambda b: (0, 0)),
            pl.BlockSpec((_N, _K), lambda b: (0, 0)),
            pl.BlockSpec((_N, _TAIL), lambda b: (0, 0)),
        ],
        out_shape=[
            jax.ShapeDtypeStruct((_N, _K), jnp.float32),
            jax.ShapeDtypeStruct((_N, _K), jnp.int32),
            jax.ShapeDtypeStruct((_N, _TAIL), jnp.float32),
        ],
        scratch_shapes=[
            pltpu.VMEM((_N, 1), jnp.float32),
            pltpu.VMEM((_N, _K), jnp.float32),
            pltpu.VMEM((_N, _K), jnp.int32),
        ],
    )(cls)

    classes = pl.pallas_call(
        _scatter_body,
        grid_spec=pltpu.PrefetchScalarGridSpec(
            num_scalar_prefetch=1,
            grid=(1,),
            in_specs=[
                pl.BlockSpec((_N, _K), lambda s, idx: (0, 0)),
                pl.BlockSpec((_N, _K), lambda s, idx: (0, 0)),
                pl.BlockSpec(memory_space=pl.ANY),
            ],
            out_specs=pl.BlockSpec(memory_space=pl.ANY),
            scratch_shapes=[
                pltpu.VMEM((_N, _WIN), jnp.float32),
                pltpu.VMEM((_N, _WIN), jnp.float32),
                pltpu.SemaphoreType.DMA,
                pltpu.SemaphoreType.DMA,
            ],
        ),
        out_shape=jax.ShapeDtypeStruct((_N, _C), jnp.float32),
        input_output_aliases={3: 0},
    )(outi, outi, outv, z)

    classes = jax.lax.dynamic_update_slice(classes, tail, (0, _TSTART))

    return (normu, classes)


# speedup vs baseline: 2.9122x; 1.4244x over previous
import functools

import jax
import jax.numpy as jnp
from jax import lax
from jax.experimental import pallas as pl
from jax.experimental.pallas import tpu as pltpu
from jax.experimental.pallas import tpu_sc as plsc

_N = 64
_C = 100000
_K = 8
_INV_T = 0.5
_NB = 8
_BLK = 12800
_NEG = float("-inf")
_IMAX = 2**31 - 1
_WIN = 128
_G = _N // 8


_TAIL = _C % _WIN
_TSTART = _C - _TAIL


def _stage1(x_ref, outv_ref, outi_ref, tail_ref, zc_ref, sum_ref, runv_ref,
            runi_ref):
    zc_ref[...] = jnp.zeros((_N, _BLK), jnp.float32)
    b = pl.program_id(0)

    @pl.when(b == 0)
    def _init():
        sum_ref[...] = jnp.zeros_like(sum_ref)
        runv_ref[...] = jnp.full((_N, _K), _NEG, jnp.float32)
        runi_ref[...] = jnp.full((_N, _K), _IMAX, jnp.int32)

    x = x_ref[...]
    col = jax.lax.broadcasted_iota(jnp.int32, (_N, _BLK), 1) + b * _BLK
    x = jnp.where(col < _C, x, _NEG)
    e = jnp.exp(x * _INV_T)
    sum_ref[...] += jnp.sum(e, axis=1, keepdims=True)

    bvs, bis = [], []
    for i in range(_K):
        m = jnp.max(x, axis=1, keepdims=True)
        hit = x == m
        idx = jnp.min(jnp.where(hit, col, _IMAX), axis=1, keepdims=True)
        bvs.append(m)
        bis.append(idx)
        x = jnp.where(hit & (col == idx), _NEG, x)

    V = jnp.concatenate([runv_ref[...]] + bvs, axis=1)
    I = jnp.concatenate([runi_ref[...]] + bis, axis=1)
    nv, ni = [], []
    for i in range(_K):
        m = jnp.max(V, axis=1, keepdims=True)
        hit = V == m
        idx = jnp.min(jnp.where(hit, I, _IMAX), axis=1, keepdims=True)
        nv.append(m)
        ni.append(idx)
        V = jnp.where(hit & (I == idx), _NEG, V)
    runv_ref[...] = jnp.concatenate(nv, axis=1)
    runi_ref[...] = jnp.concatenate(ni, axis=1)

    @pl.when(b == _NB - 1)
    def _final():
        topv = runv_ref[...]
        topi = runi_ref[...]
        denom = sum_ref[...]
        vals = []
        for i in range(_K):
            e = jnp.exp(topv[:, i:i + 1] * _INV_T)
            vals.append(e / denom)
            outv_ref[:, i:i + 1] = vals[-1]
            denom = denom - e
        outi_ref[...] = topi
        jt = jax.lax.broadcasted_iota(jnp.int32, (_N, _TAIL), 1) + _TSTART
        tacc = jnp.zeros((_N, _TAIL), jnp.float32)
        for i in range(_K):
            tacc = jnp.where(topi[:, i:i + 1] == jt, vals[i], tacc)
        tail_ref[...] = tacc


_NW = 32
_RPW = _N // _NW
_CHUNK = 20000
_NCH = _C // _CHUNK
_VPC = _CHUNK // 16
_NSEM = 4


def _sc_zero_body(out_hbm, zbuf, *sems):
    wid = lax.axis_index("s") * 2 + lax.axis_index("c")
    r0 = wid * _RPW

    def _zero(i, carry):
        zbuf[pl.ds(i * 16, 16)] = jnp.zeros((16,), jnp.float32)
        return carry

    lax.fori_loop(0, _VPC, _zero, 0)

    pending = [None] * _NSEM
    for t, (r, c) in enumerate([(r, c) for r in range(_RPW)
                                for c in range(_NCH)]):
        b = t % _NSEM
        if pending[b] is not None:
            pending[b].wait()
        pending[b] = pltpu.async_copy(
            zbuf, out_hbm.at[r0 + r, pl.ds(c * _CHUNK, _CHUNK)], sems[b])
    for cp in pending:
        if cp is not None:
            cp.wait()


@functools.lru_cache(maxsize=1)
def _make_sc_zero():
    mesh = plsc.VectorSubcoreMesh(
        core_axis_name="c", subcore_axis_name="s", num_cores=2,
        num_subcores=16)
    return pl.kernel(
        _sc_zero_body,
        out_type=jax.ShapeDtypeStruct((_N, _C), jnp.float32),
        mesh=mesh,
        scratch_types=[pltpu.VMEM((_CHUNK,), jnp.float32)]
        + [pltpu.SemaphoreType.DMA] * _NSEM,
        compiler_params=pltpu.CompilerParams(
            use_tc_tiling_on_sc=False, needs_layout_passes=False),
    )


def _scatter_body(idx_ref, idxv_ref, v_ref, z_ref, o_ref,
                  bufa, bufb, sema, semb):
    del z_ref
    idxs = idxv_ref[...]
    vals = v_ref[...]
    jota = jax.lax.broadcasted_iota(jnp.int32, (8, _WIN), 1)

    def do_pair(i, buf, sem, wait_first):
        if wait_first:
            for g in range(_G):
                pltpu.make_async_copy(
                    buf.at[pl.ds(g * 8, 8), :],
                    o_ref.at[pl.ds(g * 8, 8), pl.ds(0, _WIN)],
                    sem).wait()
        copies = []
        for g in range(_G):
            base = jnp.minimum(idx_ref[g * 8 + i // _K, i % _K] // _WIN,
                               _C // _WIN - 1) * _WIN
            win = base + jota
            acc = jnp.zeros((8, _WIN), jnp.float32)
            for k2 in range(_K):
                hit = idxs[g * 8:(g + 1) * 8, k2:k2 + 1] == win
                acc = jnp.where(hit, vals[g * 8:(g + 1) * 8, k2:k2 + 1], acc)
            buf[pl.ds(g * 8, 8), :] = acc
            copies.append((g, base))
        for g, base in copies:
            pltpu.make_async_copy(
                buf.at[pl.ds(g * 8, 8), :],
                o_ref.at[pl.ds(g * 8, 8), pl.ds(base, _WIN)],
                sem).start()

    do_pair(0, bufa, sema, False)
    do_pair(1, bufb, semb, False)

    def step(j, carry):
        do_pair(2 * j, bufa, sema, True)
        do_pair(2 * j + 1, bufb, semb, True)
        return carry

    lax.fori_loop(1, (_K * 8) // 2, step, 0)

    for buf, sem in ((bufa, sema), (bufb, semb)):
        for g in range(_G):
            pltpu.make_async_copy(
                buf.at[pl.ds(g * 8, 8), :],
                o_ref.at[pl.ds(g * 8, 8), pl.ds(0, _WIN)],
                sem).wait()


def kernel(normu, cls):
    outv, outi, tail, z = pl.pallas_call(
        _stage1,
        grid=(_NB,),
        in_specs=[pl.BlockSpec((_N, _BLK), lambda b: (0, b))],
        out_specs=[
            pl.BlockSpec((_N, _K), lambda b: (0, 0)),
            pl.BlockSpec((_N, _K), lambda b: (0, 0)),
            pl.BlockSpec((_N, _TAIL), lambda b: (0, 0)),
            pl.BlockSpec((_N, _BLK), lambda b: (0, b)),
        ],
        out_shape=[
            jax.ShapeDtypeStruct((_N, _K), jnp.float32),
            jax.ShapeDtypeStruct((_N, _K), jnp.int32),
            jax.ShapeDtypeStruct((_N, _TAIL), jnp.float32),
            jax.ShapeDtypeStruct((_N, _C), jnp.float32),
        ],
        scratch_shapes=[
            pltpu.VMEM((_N, 1), jnp.float32),
            pltpu.VMEM((_N, _K), jnp.float32),
            pltpu.VMEM((_N, _K), jnp.int32),
        ],
    )(cls)

    classes = pl.pallas_call(
        _scatter_body,
        grid_spec=pltpu.PrefetchScalarGridSpec(
            num_scalar_prefetch=1,
            grid=(1,),
            in_specs=[
                pl.BlockSpec((_N, _K), lambda s, idx: (0, 0)),
                pl.BlockSpec((_N, _K), lambda s, idx: (0, 0)),
                pl.BlockSpec(memory_space=pl.ANY),
            ],
            out_specs=pl.BlockSpec(memory_space=pl.ANY),
            scratch_shapes=[
                pltpu.VMEM((_N, _WIN), jnp.float32),
                pltpu.VMEM((_N, _WIN), jnp.float32),
                pltpu.SemaphoreType.DMA,
                pltpu.SemaphoreType.DMA,
            ],
        ),
        out_shape=jax.ShapeDtypeStruct((_N, _C), jnp.float32),
        input_output_aliases={3: 0},
    )(outi, outi, outv, z)

    classes = jax.lax.dynamic_update_slice(classes, tail, (0, _TSTART))

    return (normu, classes)


# speedup vs baseline: 3.0593x; 1.0505x over previous
import jax
import jax.numpy as jnp
from jax.experimental import pallas as pl
from jax.experimental.pallas import tpu as pltpu

_N = 64
_C = 100000
_K = 8
_INV_T = 0.5
_NB = 8
_BLK = 12800
_NEG = float("-inf")
_IMAX = 2**31 - 1


def _stage1(x_ref, outv_ref, outi_ref, sum_ref, runv_ref, runi_ref):
    b = pl.program_id(0)

    @pl.when(b == 0)
    def _init():
        sum_ref[...] = jnp.zeros_like(sum_ref)
        runv_ref[...] = jnp.full((_N, _K), _NEG, jnp.float32)
        runi_ref[...] = jnp.full((_N, _K), _IMAX, jnp.int32)

    x = x_ref[...]
    col = jax.lax.broadcasted_iota(jnp.int32, (_N, _BLK), 1) + b * _BLK
    x = jnp.where(col < _C, x, _NEG)
    e = jnp.exp(x * _INV_T)
    sum_ref[...] += jnp.sum(e, axis=1, keepdims=True)

    bvs, bis = [], []
    for i in range(_K):
        m = jnp.max(x, axis=1, keepdims=True)
        hit = x == m
        idx = jnp.min(jnp.where(hit, col, _IMAX), axis=1, keepdims=True)
        bvs.append(m)
        bis.append(idx)
        x = jnp.where(hit & (col == idx), _NEG, x)

    V = jnp.concatenate([runv_ref[...]] + bvs, axis=1)
    I = jnp.concatenate([runi_ref[...]] + bis, axis=1)
    nv, ni = [], []
    for i in range(_K):
        m = jnp.max(V, axis=1, keepdims=True)
        hit = V == m
        idx = jnp.min(jnp.where(hit, I, _IMAX), axis=1, keepdims=True)
        nv.append(m)
        ni.append(idx)
        V = jnp.where(hit & (I == idx), _NEG, V)
    runv_ref[...] = jnp.concatenate(nv, axis=1)
    runi_ref[...] = jnp.concatenate(ni, axis=1)

    @pl.when(b == _NB - 1)
    def _final():
        topv = runv_ref[...]
        denom = sum_ref[...]
        for i in range(_K):
            e = jnp.exp(topv[:, i:i + 1] * _INV_T)
            outv_ref[:, i:i + 1] = e / denom
            denom = denom - e
        outi_ref[...] = runi_ref[...]


def _stage2(outi_ref, outv_ref, o_ref):
    b = pl.program_id(0)
    col = jax.lax.broadcasted_iota(jnp.int32, (_N, _BLK), 1) + b * _BLK
    acc = jnp.zeros((_N, _BLK), jnp.float32)
    for i in range(_K):
        acc = jnp.where(col == outi_ref[:, i:i + 1], outv_ref[:, i:i + 1], acc)
    o_ref[...] = acc


def kernel(normu, cls):
    outv, outi = pl.pallas_call(
        _stage1,
        grid=(_NB,),
        in_specs=[pl.BlockSpec((_N, _BLK), lambda b: (0, b))],
        out_specs=[
            pl.BlockSpec((_N, _K), lambda b: (0, 0)),
            pl.BlockSpec((_N, _K), lambda b: (0, 0)),
        ],
        out_shape=[
            jax.ShapeDtypeStruct((_N, _K), jnp.float32),
            jax.ShapeDtypeStruct((_N, _K), jnp.int32),
        ],
        scratch_shapes=[
            pltpu.VMEM((_N, 1), jnp.float32),
            pltpu.VMEM((_N, _K), jnp.float32),
            pltpu.VMEM((_N, _K), jnp.int32),
        ],
    )(cls)

    classes = pl.pallas_call(
        _stage2,
        grid=(_NB,),
        in_specs=[
            pl.BlockSpec((_N, _K), lambda b: (0, 0)),
            pl.BlockSpec((_N, _K), lambda b: (0, 0)),
        ],
        out_specs=pl.BlockSpec((_N, _BLK), lambda b: (0, b)),
        out_shape=jax.ShapeDtypeStruct((_N, _C), jnp.float32),
    )(outi, outv)

    return (normu, classes)
